# baseline (speedup 1.0000x reference)
import functools

import numpy as np
import jax
import jax.numpy as jnp
from jax import lax
from jax.experimental import pallas as pl
from jax.experimental.pallas import tpu as pltpu
from jax.experimental.pallas import tpu_sc as plsc

_LANES = 16


def _positional_encoding_np(d_model, length):
    pos = np.arange(length, dtype=np.float32)[:, None]
    div = np.exp(
        np.arange(0, d_model, 2, dtype=np.float32) * (-np.log(10000.0) / d_model)
    )
    pe = np.zeros((length, d_model), dtype=np.float32)
    pe[:, 0::2] = np.sin(pos * div)
    pe[:, 1::2] = np.cos(pos * div)
    return pe


def kernel(x, table):
    B, L = x.shape
    V, D = table.shape
    N = B * L

    NC, NS = 2, 16
    NW = NC * NS
    per_w = N // NW
    assert per_w * NW == N

    seq_per_chunk = 2
    RPC = seq_per_chunk * L
    NCH = per_w // RPC
    assert NCH * RPC == per_w
    G = 80
    NG = RPC // G
    assert NG * G == RPC

    xf = x.reshape(N).astype(jnp.int32)
    pe = jnp.asarray(_positional_encoding_np(D, L))

    mesh = plsc.VectorSubcoreMesh(core_axis_name="c", subcore_axis_name="s")

    @functools.partial(
        pl.kernel,
        mesh=mesh,
        compiler_params=pltpu.CompilerParams(use_tc_tiling_on_sc=False),
        out_type=jax.ShapeDtypeStruct((N, D), jnp.float32),
        scratch_types=[
            pltpu.VMEM((L, D), jnp.float32),
            pltpu.VMEM((RPC,), jnp.int32),
            pltpu.VMEM((RPC, D), jnp.float32),
            pltpu.SemaphoreType.DMA,
        ],
    )
    def run(xf_hbm, table_hbm, pe_hbm, out_hbm, pe_v, idx_v, rows_v, sem):
        wid = lax.axis_index("s") * NC + lax.axis_index("c")
        base = wid * per_w
        pltpu.sync_copy(pe_hbm, pe_v)

        def chunk_body(g, carry):
            rbase = base + g * RPC
            pltpu.sync_copy(xf_hbm.at[pl.ds(rbase, RPC)], idx_v)
            copies = [
                pltpu.async_copy(
                    table_hbm.at[idx_v.at[pl.ds(k * G, G)]],
                    rows_v.at[pl.ds(k * G, G)],
                    sem,
                )
                for k in range(NG)
            ]
            for c in copies:
                c.wait()

            def add_body(j, carry2):
                for s_ in range(seq_per_chunk):
                    r = s_ * L + j
                    for q in range(D // _LANES):
                        sl = pl.ds(q * _LANES, _LANES)
                        rows_v[r, sl] = rows_v[r, sl] + pe_v[j, sl]
                return carry2

            lax.fori_loop(0, L, add_body, 0)
            pltpu.sync_copy(rows_v, out_hbm.at[pl.ds(rbase, RPC)])
            return carry

        lax.fori_loop(0, NCH, chunk_body, 0)

    out = run(xf, table, pe)
    return out.reshape(B, L, D)
